# baseline (speedup 1.0000x reference)
import functools

import jax
import jax.numpy as jnp
from jax import lax
from jax.experimental import pallas as pl
from jax.experimental.pallas import tpu as pltpu, tpu_sc as plsc

_B, _L, _D = 16, 4096, 256
_H, _O = 512, 128
_R = 128
_NSEG = _D // 16
_NW = 32


def _pool_body(x_hbm, l_hbm, p_hbm,
               len_v, buf0, buf1, stage, partial, shared, sem0, sem1):
    cid = lax.axis_index("c")
    sid = lax.axis_index("s")
    wid = sid * 2 + cid
    zv = jnp.zeros((16,), jnp.float32)

    pltpu.sync_copy(l_hbm, len_v.at[pl.ds(0, _B)])
    lens_s = [len_v[pl.ds(b, 16)][0] for b in range(_B)]
    ncs_s = [lax.shift_right_logical(l + (_R - 1), 7) for l in lens_s]
    cs_s = []
    run = jnp.int32(0)
    for b in range(_B):
        run = run + ncs_s[b]
        cs_s.append(run)
    total = run

    def chunk_info(g):
        b = jnp.int32(0)
        excl = jnp.int32(0)
        lenb = lens_s[0]
        for bb in range(1, _B):
            cond = g >= cs_s[bb - 1]
            b = jnp.where(cond, jnp.int32(bb), b)
            excl = jnp.where(cond, cs_s[bb - 1], excl)
            lenb = jnp.where(cond, lens_s[bb], lenb)
        c0 = (g - excl) * _R
        return b, c0, lenb

    def start_fetch(g, buf, sem):
        b, c0, _ = chunk_info(g)
        pltpu.make_async_copy(
            x_hbm.at[b, pl.ds(c0, _R), :], buf, sem).start()

    def wait_fetch(buf, sem):
        pltpu.make_async_copy(
            x_hbm.at[0, pl.ds(0, _R), :], buf, sem).wait()

    def accum_chunk(g, buf):
        b, c0, lenb = chunk_info(g)
        nrows = jnp.minimum(lenb - c0, _R)
        ngr = lax.shift_right_logical(nrows, 3)

        def grp(k, a):
            base = k * 8
            for rr in range(8):
                r = base + rr
                a = tuple(a[d] + buf[r, 16 * d:16 * (d + 1)]
                          for d in range(_NSEG))
            return a

        accs = lax.fori_loop(0, ngr, grp, (zv,) * _NSEG)

        def tail(r, a):
            return tuple(a[d] + buf[r, 16 * d:16 * (d + 1)]
                         for d in range(_NSEG))

        accs = lax.fori_loop(ngr * 8, nrows, tail, accs)
        for d in range(_NSEG):
            plsc.addupdate(partial.at[b, 16 * d:16 * (d + 1)], accs[d])

    for t in range(_B):
        for d in range(_NSEG):
            partial[t, 16 * d:16 * (d + 1)] = zv

    nmine = lax.shift_right_logical(jnp.maximum(total - wid + 31, 0), 5)
    npairs = lax.shift_right_logical(nmine + 1, 1)

    @pl.when(nmine > 0)
    def _():
        start_fetch(wid, buf0, sem0)

    def pair_body(p, carry):
        i1 = 2 * p + 1
        g0 = wid + 64 * p
        g1 = g0 + 32
        wait_fetch(buf0, sem0)

        @pl.when(i1 < nmine)
        def _():
            start_fetch(g1, buf1, sem1)

        accum_chunk(g0, buf0)

        @pl.when(i1 < nmine)
        def _():
            wait_fetch(buf1, sem1)

            @pl.when(i1 + 1 < nmine)
            def _():
                start_fetch(g0 + 64, buf0, sem0)

            accum_chunk(g1, buf1)

        return carry

    lax.fori_loop(0, npairs, pair_body, 0)
    pltpu.sync_copy(partial, shared.at[sid])
    plsc.subcore_barrier()
    accs = [zv] * _NSEG
    for s in range(16):
        pltpu.sync_copy(shared.at[s, pl.ds(sid, 1)], stage)
        for d in range(_NSEG):
            accs[d] = accs[d] + stage[0, 16 * d:16 * (d + 1)]
    for d in range(_NSEG):
        stage[0, 16 * d:16 * (d + 1)] = accs[d]
    pltpu.sync_copy(stage, p_hbm.at[cid, pl.ds(sid, 1)])


_pool = pl.kernel(
    _pool_body,
    out_type=jax.ShapeDtypeStruct((2, _B, _D), jnp.float32),
    mesh=plsc.VectorSubcoreMesh(core_axis_name="c", subcore_axis_name="s"),
    scratch_types=[
        pltpu.VMEM((2 * _B,), jnp.int32),
        pltpu.VMEM((_R, _D), jnp.float32),
        pltpu.VMEM((_R, _D), jnp.float32),
        pltpu.VMEM((1, _D), jnp.float32),
        pltpu.VMEM((_B, _D), jnp.float32),
        pltpu.VMEM_SHARED((16, _B, _D), jnp.float32),
        pltpu.SemaphoreType.DMA,
        pltpu.SemaphoreType.DMA,
    ],
)


_RC = 512
_NBUF = 4
_NSC = 11
_NT = _B + (_B - _NSC)


def _tcpool_body(lens_ref, x2_hbm, x1_hbm, o_ref, buf, nc_tbl,
                 sem0, sem1, sem2, sem3):
    sems = (sem0, sem1, sem2, sem3)
    o_ref[...] = jnp.zeros((_NT, 8, _D), jnp.float32)

    total = jnp.int32(0)
    for b in range(_NT):
        nc = lax.shift_right_logical(lens_ref[b] + (_RC - 1), 9)
        nc_tbl[b] = nc
        total = total + nc

    def advance(t, c):
        nxt = (c + 1) >= nc_tbl[jnp.minimum(t, _NT - 1)]
        t2 = jnp.where(nxt, t + 1, t)
        c2 = jnp.where(nxt, 0, c + 1)
        return t2, c2

    def start_fetch(t, c, j):
        tc = jnp.minimum(t, _NT - 1)

        @pl.when(tc < _B)
        def _():
            pltpu.make_async_copy(
                x2_hbm.at[jnp.minimum(tc, _B - 1), pl.ds(c * _RC, _RC), :],
                buf.at[j], sems[j]).start()

        @pl.when(tc >= _B)
        def _():
            pltpu.make_async_copy(
                x1_hbm.at[jnp.minimum(tc - _B + _NSC, _B - 1),
                          pl.ds(c * _RC, _RC), :],
                buf.at[j], sems[j]).start()

    def wait_fetch(j):
        pltpu.make_async_copy(
            x2_hbm.at[0, pl.ds(0, _RC), :], buf.at[j], sems[j]).wait()

    def process(t, c, j):
        tc = jnp.minimum(t, _NT - 1)
        lent = lens_ref[tc]
        c0 = c * _RC
        x = buf[j]

        @pl.when(c0 + _RC <= lent)
        def _():
            a8 = jnp.sum(x.reshape(8, _RC // 64, 8, _D), axis=0)
            acc = jnp.sum(a8, axis=0)
            o_ref[pl.ds(tc, 1)] += acc.reshape(1, 8, _D)

        @pl.when(c0 + _RC > lent)
        def _():
            rows = lax.broadcasted_iota(jnp.int32, (_RC, 1), 0) + c0
            mask = (rows < lent).astype(jnp.float32)
            a8 = jnp.sum((x * mask).reshape(8, _RC // 64, 8, _D), axis=0)
            acc = jnp.sum(a8, axis=0)
            o_ref[pl.ds(tc, 1)] += acc.reshape(1, 8, _D)

    tf = jnp.int32(0)
    cf = jnp.int32(0)
    for j in range(_NBUF - 1):
        @pl.when(j < total)
        def _():
            start_fetch(tf, cf, j)

        tf, cf = advance(tf, cf)

    nouter = lax.shift_right_logical(total + (_NBUF - 1), 2)

    def outer(p, carry):
        tf, cf, tp, cp = carry
        for j in range(_NBUF):
            g = p * _NBUF + j

            @pl.when(g < total)
            def _():
                wait_fetch(j)

                @pl.when(g + (_NBUF - 1) < total)
                def _():
                    start_fetch(tf, cf, (j + _NBUF - 1) % _NBUF)

                process(tp, cp, j)

            tf, cf = advance(tf, cf)
            tp, cp = advance(tp, cp)

        return tf, cf, tp, cp

    z = jnp.int32(0)
    lax.fori_loop(0, nouter, outer, (tf, cf, z, z))


_tcpool = pl.pallas_call(
    _tcpool_body,
    in_specs=[pl.BlockSpec(memory_space=pltpu.SMEM),
              pl.BlockSpec(memory_space=pl.ANY),
              pl.BlockSpec(memory_space=pl.ANY)],
    out_specs=pl.BlockSpec(memory_space=pltpu.VMEM),
    out_shape=jax.ShapeDtypeStruct((_NT, 8, _D), jnp.float32),
    scratch_shapes=[
        pltpu.VMEM((_NBUF, _RC, _D), jnp.float32),
        pltpu.SMEM((_NT,), jnp.int32),
        pltpu.SemaphoreType.DMA,
        pltpu.SemaphoreType.DMA,
        pltpu.SemaphoreType.DMA,
        pltpu.SemaphoreType.DMA,
    ],
)


def _mlp_body(p_ref, s_ref, l1_ref, l2_ref,
              w1_ref, b1_ref, w2_ref, b2_ref, o_ref):
    s = jnp.sum(s_ref[...], axis=1)
    s1_tail = s[_B:]
    e1_sum = (p_ref[0] + p_ref[1]) + jnp.concatenate(
        [jnp.zeros((_NSC, _D), jnp.float32), s1_tail], axis=0)
    e1 = e1_sum / l1_ref[...]
    e2 = s[:_B] / l2_ref[...]
    f = jnp.concatenate([e1, e2, jnp.abs(e1 - e2), e1 * e2], axis=1)
    h = jnp.dot(f, w1_ref[...], preferred_element_type=jnp.float32)
    h = jnp.maximum(h + b1_ref[...], 0.0)
    o = jnp.dot(h, w2_ref[...], preferred_element_type=jnp.float32)
    o_ref[...] = o + b2_ref[...]


_mlp = pl.pallas_call(
    _mlp_body,
    out_shape=jax.ShapeDtypeStruct((_B, _O), jnp.float32),
)


def kernel(X1, x1_lengths, X2, x2_lengths, W1, b1, W2, b2):
    sc_mask = jnp.arange(_B, dtype=jnp.int32) < _NSC
    l1_sc = jnp.where(sc_mask, x1_lengths, 0)
    p1 = _pool(X1, l1_sc)
    lens_tc = jnp.concatenate([x2_lengths, x1_lengths[_NSC:]])
    s = _tcpool(lens_tc, X2, X1)
    l1f = x1_lengths.astype(jnp.float32).reshape(_B, 1)
    l2f = x2_lengths.astype(jnp.float32).reshape(_B, 1)
    return _mlp(p1, s, l1f, l2f,
                W1, b1.reshape(1, _H), W2, b2.reshape(1, _O))


# speedup vs baseline: 1.0044x; 1.0044x over previous
import functools

import jax
import jax.numpy as jnp
from jax import lax
from jax.experimental import pallas as pl
from jax.experimental.pallas import tpu as pltpu, tpu_sc as plsc

_B, _L, _D = 16, 4096, 256
_H, _O = 512, 128
_R = 128
_NSEG = _D // 16
_NW = 32


def _pool_body(x_hbm, l_hbm, p_hbm,
               len_v, buf0, buf1, stage, partial, shared, sem0, sem1):
    cid = lax.axis_index("c")
    sid = lax.axis_index("s")
    wid = sid * 2 + cid
    zv = jnp.zeros((16,), jnp.float32)

    pltpu.sync_copy(l_hbm, len_v.at[pl.ds(0, _B)])
    lens_s = [len_v[pl.ds(b, 16)][0] for b in range(_B)]
    ncs_s = [lax.shift_right_logical(l + (_R - 1), 7) for l in lens_s]
    cs_s = []
    run = jnp.int32(0)
    for b in range(_B):
        run = run + ncs_s[b]
        cs_s.append(run)
    total = run

    def chunk_info(g):
        b = jnp.int32(0)
        excl = jnp.int32(0)
        lenb = lens_s[0]
        for bb in range(1, _B):
            cond = g >= cs_s[bb - 1]
            b = jnp.where(cond, jnp.int32(bb), b)
            excl = jnp.where(cond, cs_s[bb - 1], excl)
            lenb = jnp.where(cond, lens_s[bb], lenb)
        c0 = (g - excl) * _R
        return b, c0, lenb

    def start_fetch(g, buf, sem):
        b, c0, _ = chunk_info(g)
        pltpu.make_async_copy(
            x_hbm.at[b, pl.ds(c0, _R), :], buf, sem).start()

    def wait_fetch(buf, sem):
        pltpu.make_async_copy(
            x_hbm.at[0, pl.ds(0, _R), :], buf, sem).wait()

    def accum_chunk(g, buf):
        b, c0, lenb = chunk_info(g)
        nrows = jnp.minimum(lenb - c0, _R)
        ngr = lax.shift_right_logical(nrows, 3)

        def grp(k, a):
            base = k * 8
            for rr in range(8):
                r = base + rr
                a = tuple(a[d] + buf[r, 16 * d:16 * (d + 1)]
                          for d in range(_NSEG))
            return a

        accs = lax.fori_loop(0, ngr, grp, (zv,) * _NSEG)

        def tail(r, a):
            return tuple(a[d] + buf[r, 16 * d:16 * (d + 1)]
                         for d in range(_NSEG))

        accs = lax.fori_loop(ngr * 8, nrows, tail, accs)
        for d in range(_NSEG):
            plsc.addupdate(partial.at[b, 16 * d:16 * (d + 1)], accs[d])

    for t in range(_B):
        for d in range(_NSEG):
            partial[t, 16 * d:16 * (d + 1)] = zv

    nmine = lax.shift_right_logical(jnp.maximum(total - wid + 31, 0), 5)
    npairs = lax.shift_right_logical(nmine + 1, 1)

    @pl.when(nmine > 0)
    def _():
        start_fetch(wid, buf0, sem0)

    def pair_body(p, carry):
        i1 = 2 * p + 1
        g0 = wid + 64 * p
        g1 = g0 + 32
        wait_fetch(buf0, sem0)

        @pl.when(i1 < nmine)
        def _():
            start_fetch(g1, buf1, sem1)

        accum_chunk(g0, buf0)

        @pl.when(i1 < nmine)
        def _():
            wait_fetch(buf1, sem1)

            @pl.when(i1 + 1 < nmine)
            def _():
                start_fetch(g0 + 64, buf0, sem0)

            accum_chunk(g1, buf1)

        return carry

    lax.fori_loop(0, npairs, pair_body, 0)
    pltpu.sync_copy(partial, shared.at[sid])
    plsc.subcore_barrier()
    accs = [zv] * _NSEG
    for s in range(16):
        pltpu.sync_copy(shared.at[s, pl.ds(sid, 1)], stage)
        for d in range(_NSEG):
            accs[d] = accs[d] + stage[0, 16 * d:16 * (d + 1)]
    for d in range(_NSEG):
        stage[0, 16 * d:16 * (d + 1)] = accs[d]
    pltpu.sync_copy(stage, p_hbm.at[cid, pl.ds(sid, 1)])


_pool = pl.kernel(
    _pool_body,
    out_type=jax.ShapeDtypeStruct((2, _B, _D), jnp.float32),
    mesh=plsc.VectorSubcoreMesh(core_axis_name="c", subcore_axis_name="s"),
    scratch_types=[
        pltpu.VMEM((2 * _B,), jnp.int32),
        pltpu.VMEM((_R, _D), jnp.float32),
        pltpu.VMEM((_R, _D), jnp.float32),
        pltpu.VMEM((1, _D), jnp.float32),
        pltpu.VMEM((_B, _D), jnp.float32),
        pltpu.VMEM_SHARED((16, _B, _D), jnp.float32),
        pltpu.SemaphoreType.DMA,
        pltpu.SemaphoreType.DMA,
    ],
)


_RC = 512
_NBUF = 4
_NSC = 11
_NT = _B + (_B - _NSC)


def _tcpool_body(lens_ref, x2_hbm, x2b_hbm, x1_hbm, o_ref, buf, nc_tbl,
                 sem0, sem1, sem2, sem3):
    sems = (sem0, sem1, sem2, sem3)
    o_ref[...] = jnp.zeros((_NT, 8, _D), jnp.float32)

    total = jnp.int32(0)
    for b in range(_NT):
        nc = lax.shift_right_logical(lens_ref[b] + (_RC - 1), 9)
        nc_tbl[b] = nc
        total = total + nc

    def advance(t, c):
        nxt = (c + 1) >= nc_tbl[jnp.minimum(t, _NT - 1)]
        t2 = jnp.where(nxt, t + 1, t)
        c2 = jnp.where(nxt, 0, c + 1)
        return t2, c2

    def start_fetch(t, c, j):
        tc = jnp.minimum(t, _NT - 1)

        @pl.when(tc < _B)
        def _():
            xr = x2_hbm if (j % 2 == 0) else x2b_hbm
            pltpu.make_async_copy(
                xr.at[jnp.minimum(tc, _B - 1), pl.ds(c * _RC, _RC), :],
                buf.at[j], sems[j]).start()

        @pl.when(tc >= _B)
        def _():
            pltpu.make_async_copy(
                x1_hbm.at[jnp.minimum(tc - _B + _NSC, _B - 1),
                          pl.ds(c * _RC, _RC), :],
                buf.at[j], sems[j]).start()

    def wait_fetch(j):
        pltpu.make_async_copy(
            x2_hbm.at[0, pl.ds(0, _RC), :], buf.at[j], sems[j]).wait()

    def process(t, c, j):
        tc = jnp.minimum(t, _NT - 1)
        lent = lens_ref[tc]
        c0 = c * _RC
        x = buf[j]

        @pl.when(c0 + _RC <= lent)
        def _():
            a8 = jnp.sum(x.reshape(8, _RC // 64, 8, _D), axis=0)
            acc = jnp.sum(a8, axis=0)
            o_ref[pl.ds(tc, 1)] += acc.reshape(1, 8, _D)

        @pl.when(c0 + _RC > lent)
        def _():
            rows = lax.broadcasted_iota(jnp.int32, (_RC, 1), 0) + c0
            mask = (rows < lent).astype(jnp.float32)
            a8 = jnp.sum((x * mask).reshape(8, _RC // 64, 8, _D), axis=0)
            acc = jnp.sum(a8, axis=0)
            o_ref[pl.ds(tc, 1)] += acc.reshape(1, 8, _D)

    tf = jnp.int32(0)
    cf = jnp.int32(0)
    for j in range(_NBUF - 1):
        @pl.when(j < total)
        def _():
            start_fetch(tf, cf, j)

        tf, cf = advance(tf, cf)

    nouter = lax.shift_right_logical(total + (_NBUF - 1), 2)

    def outer(p, carry):
        tf, cf, tp, cp = carry
        for j in range(_NBUF):
            g = p * _NBUF + j

            @pl.when(g < total)
            def _():
                wait_fetch(j)

                @pl.when(g + (_NBUF - 1) < total)
                def _():
                    start_fetch(tf, cf, (j + _NBUF - 1) % _NBUF)

                process(tp, cp, j)

            tf, cf = advance(tf, cf)
            tp, cp = advance(tp, cp)

        return tf, cf, tp, cp

    z = jnp.int32(0)
    lax.fori_loop(0, nouter, outer, (tf, cf, z, z))


_tcpool = pl.pallas_call(
    _tcpool_body,
    in_specs=[pl.BlockSpec(memory_space=pltpu.SMEM),
              pl.BlockSpec(memory_space=pl.ANY),
              pl.BlockSpec(memory_space=pl.ANY),
              pl.BlockSpec(memory_space=pl.ANY)],
    out_specs=pl.BlockSpec(memory_space=pltpu.VMEM),
    out_shape=jax.ShapeDtypeStruct((_NT, 8, _D), jnp.float32),
    scratch_shapes=[
        pltpu.VMEM((_NBUF, _RC, _D), jnp.float32),
        pltpu.SMEM((_NT,), jnp.int32),
        pltpu.SemaphoreType.DMA,
        pltpu.SemaphoreType.DMA,
        pltpu.SemaphoreType.DMA,
        pltpu.SemaphoreType.DMA,
    ],
)


def _mlp_body(p_ref, s_ref, l1_ref, l2_ref,
              w1_ref, b1_ref, w2_ref, b2_ref, o_ref):
    s = jnp.sum(s_ref[...], axis=1)
    s1_tail = s[_B:]
    e1_sum = (p_ref[0] + p_ref[1]) + jnp.concatenate(
        [jnp.zeros((_NSC, _D), jnp.float32), s1_tail], axis=0)
    e1 = e1_sum / l1_ref[...]
    e2 = s[:_B] / l2_ref[...]
    f = jnp.concatenate([e1, e2, jnp.abs(e1 - e2), e1 * e2], axis=1)
    h = jnp.dot(f, w1_ref[...], preferred_element_type=jnp.float32)
    h = jnp.maximum(h + b1_ref[...], 0.0)
    o = jnp.dot(h, w2_ref[...], preferred_element_type=jnp.float32)
    o_ref[...] = o + b2_ref[...]


_mlp = pl.pallas_call(
    _mlp_body,
    out_shape=jax.ShapeDtypeStruct((_B, _O), jnp.float32),
)


def kernel(X1, x1_lengths, X2, x2_lengths, W1, b1, W2, b2):
    sc_mask = jnp.arange(_B, dtype=jnp.int32) < _NSC
    l1_sc = jnp.where(sc_mask, x1_lengths, 0)
    p1 = _pool(X1, l1_sc)
    lens_tc = jnp.concatenate([x2_lengths, x1_lengths[_NSC:]])
    s = _tcpool(lens_tc, X2, X2, X1)
    l1f = x1_lengths.astype(jnp.float32).reshape(_B, 1)
    l2f = x2_lengths.astype(jnp.float32).reshape(_B, 1)
    return _mlp(p1, s, l1f, l2f,
                W1, b1.reshape(1, _H), W2, b2.reshape(1, _O))


# speedup vs baseline: 1.1710x; 1.1659x over previous
import functools

import jax
import jax.numpy as jnp
from jax import lax
from jax.experimental import pallas as pl
from jax.experimental.pallas import tpu as pltpu, tpu_sc as plsc

_B, _L, _D = 16, 4096, 256
_H, _O = 512, 128
_R = 128
_NSEG = _D // 16
_NW = 32


def _pool_body(x_hbm, l_hbm, p_hbm,
               len_v, buf0, buf1, stage, partial, shared, sem0, sem1):
    cid = lax.axis_index("c")
    sid = lax.axis_index("s")
    wid = sid * 2 + cid
    zv = jnp.zeros((16,), jnp.float32)

    pltpu.sync_copy(l_hbm, len_v.at[pl.ds(0, _B)])
    lens_s = [len_v[pl.ds(b, 16)][0] for b in range(_B)]
    ncs_s = [lax.shift_right_logical(l + (_R - 1), 7) for l in lens_s]
    cs_s = []
    run = jnp.int32(0)
    for b in range(_B):
        run = run + ncs_s[b]
        cs_s.append(run)
    total = run

    def chunk_info(g):
        b = jnp.int32(0)
        excl = jnp.int32(0)
        lenb = lens_s[0]
        for bb in range(1, _B):
            cond = g >= cs_s[bb - 1]
            b = jnp.where(cond, jnp.int32(bb), b)
            excl = jnp.where(cond, cs_s[bb - 1], excl)
            lenb = jnp.where(cond, lens_s[bb], lenb)
        c0 = (g - excl) * _R
        return b, c0, lenb

    def start_fetch(g, buf, sem):
        b, c0, _ = chunk_info(g)
        pltpu.make_async_copy(
            x_hbm.at[b, pl.ds(c0, _R), :], buf, sem).start()

    def wait_fetch(buf, sem):
        pltpu.make_async_copy(
            x_hbm.at[0, pl.ds(0, _R), :], buf, sem).wait()

    def accum_chunk(g, buf):
        b, c0, lenb = chunk_info(g)
        nrows = jnp.minimum(lenb - c0, _R)
        ngr = lax.shift_right_logical(nrows, 3)

        def grp(k, a):
            base = k * 8
            for rr in range(8):
                r = base + rr
                a = tuple(a[d] + buf[r, 16 * d:16 * (d + 1)]
                          for d in range(_NSEG))
            return a

        accs = lax.fori_loop(0, ngr, grp, (zv,) * _NSEG)

        def tail(r, a):
            return tuple(a[d] + buf[r, 16 * d:16 * (d + 1)]
                         for d in range(_NSEG))

        accs = lax.fori_loop(ngr * 8, nrows, tail, accs)
        for d in range(_NSEG):
            plsc.addupdate(partial.at[b, 16 * d:16 * (d + 1)], accs[d])

    for t in range(_B):
        for d in range(_NSEG):
            partial[t, 16 * d:16 * (d + 1)] = zv

    nmine = lax.shift_right_logical(jnp.maximum(total - wid + 31, 0), 5)
    npairs = lax.shift_right_logical(nmine + 1, 1)

    @pl.when(nmine > 0)
    def _():
        start_fetch(wid, buf0, sem0)

    def pair_body(p, carry):
        i1 = 2 * p + 1
        g0 = wid + 64 * p
        g1 = g0 + 32
        wait_fetch(buf0, sem0)

        @pl.when(i1 < nmine)
        def _():
            start_fetch(g1, buf1, sem1)

        accum_chunk(g0, buf0)

        @pl.when(i1 < nmine)
        def _():
            wait_fetch(buf1, sem1)

            @pl.when(i1 + 1 < nmine)
            def _():
                start_fetch(g0 + 64, buf0, sem0)

            accum_chunk(g1, buf1)

        return carry

    lax.fori_loop(0, npairs, pair_body, 0)
    pltpu.sync_copy(partial, shared.at[sid])
    plsc.subcore_barrier()
    accs = [zv] * _NSEG
    for s in range(16):
        pltpu.sync_copy(shared.at[s, pl.ds(sid, 1)], stage)
        for d in range(_NSEG):
            accs[d] = accs[d] + stage[0, 16 * d:16 * (d + 1)]
    for d in range(_NSEG):
        stage[0, 16 * d:16 * (d + 1)] = accs[d]
    pltpu.sync_copy(stage, p_hbm.at[cid, pl.ds(sid, 1)])


_pool = pl.kernel(
    _pool_body,
    out_type=jax.ShapeDtypeStruct((2, _B, _D), jnp.float32),
    mesh=plsc.VectorSubcoreMesh(core_axis_name="c", subcore_axis_name="s"),
    scratch_types=[
        pltpu.VMEM((2 * _B,), jnp.int32),
        pltpu.VMEM((_R, _D), jnp.float32),
        pltpu.VMEM((_R, _D), jnp.float32),
        pltpu.VMEM((1, _D), jnp.float32),
        pltpu.VMEM((_B, _D), jnp.float32),
        pltpu.VMEM_SHARED((16, _B, _D), jnp.float32),
        pltpu.SemaphoreType.DMA,
        pltpu.SemaphoreType.DMA,
    ],
)


_RC = 512
_NBUF = 4
_NSC = 12
_NT = _B + (_B - _NSC)
_NP = _NT // 2


def _tcpool_body(lens_ref, x2_hbm, x1_hbm, o_ref, buf, nc_tbl,
                 sem0, sem1, sem2, sem3):
    sems = (sem0, sem1, sem2, sem3)
    o_ref[...] = jnp.zeros((_NT, 8, _D), jnp.float32)

    total = jnp.int32(0)
    for p in range(_NP):
        lmax = jnp.maximum(lens_ref[2 * p], lens_ref[2 * p + 1])
        nc = lax.shift_right_logical(lmax + (_RC - 1), 9)
        nc_tbl[p] = nc
        total = total + nc

    def advance(t, c):
        nxt = (c + 1) >= nc_tbl[jnp.minimum(t, _NP - 1)]
        t2 = jnp.where(nxt, t + 1, t)
        c2 = jnp.where(nxt, 0, c + 1)
        return t2, c2

    def start_fetch(t, c, j):
        tc = jnp.minimum(t, _NP - 1)

        @pl.when(tc < _B // 2)
        def _():
            pltpu.make_async_copy(
                x2_hbm.at[pl.ds(jnp.minimum(tc, _B // 2 - 1) * 2, 2),
                          pl.ds(c * _RC, _RC), :],
                buf.at[j], sems[j]).start()

        @pl.when(tc >= _B // 2)
        def _():
            b0 = (tc - _B // 2) * 2 + _NSC
            pltpu.make_async_copy(
                x1_hbm.at[pl.ds(jnp.minimum(b0, _B - 2), 2),
                          pl.ds(c * _RC, _RC), :],
                buf.at[j], sems[j]).start()

    def wait_fetch(j):
        pltpu.make_async_copy(
            x2_hbm.at[pl.ds(0, 2), pl.ds(0, _RC), :],
            buf.at[j], sems[j]).wait()

    def process(t, c, j):
        tc = jnp.minimum(t, _NP - 1)
        c0 = c * _RC
        for s in range(2):
            lent = lens_ref[2 * tc + s]
            x = buf[j, s]

            @pl.when(c0 + _RC <= lent)
            def _():
                a8 = jnp.sum(x.reshape(8, _RC // 64, 8, _D), axis=0)
                acc = jnp.sum(a8, axis=0)
                o_ref[pl.ds(2 * tc + s, 1)] += acc.reshape(1, 8, _D)

            @pl.when(jnp.logical_and(c0 < lent, lent < c0 + _RC))
            def _():
                rows = lax.broadcasted_iota(jnp.int32, (_RC, 1), 0) + c0
                mask = (rows < lent).astype(jnp.float32)
                a8 = jnp.sum((x * mask).reshape(8, _RC // 64, 8, _D), axis=0)
                acc = jnp.sum(a8, axis=0)
                o_ref[pl.ds(2 * tc + s, 1)] += acc.reshape(1, 8, _D)

    tf = jnp.int32(0)
    cf = jnp.int32(0)
    for j in range(_NBUF - 1):
        @pl.when(j < total)
        def _():
            start_fetch(tf, cf, j)

        tf, cf = advance(tf, cf)

    nouter = lax.shift_right_logical(total + (_NBUF - 1), 2)

    def outer(p, carry):
        tf, cf, tp, cp = carry
        for j in range(_NBUF):
            g = p * _NBUF + j

            @pl.when(g < total)
            def _():
                wait_fetch(j)

                @pl.when(g + (_NBUF - 1) < total)
                def _():
                    start_fetch(tf, cf, (j + _NBUF - 1) % _NBUF)

                process(tp, cp, j)

            tf, cf = advance(tf, cf)
            tp, cp = advance(tp, cp)

        return tf, cf, tp, cp

    z = jnp.int32(0)
    lax.fori_loop(0, nouter, outer, (tf, cf, z, z))


_tcpool = pl.pallas_call(
    _tcpool_body,
    in_specs=[pl.BlockSpec(memory_space=pltpu.SMEM),
              pl.BlockSpec(memory_space=pl.ANY),
              pl.BlockSpec(memory_space=pl.ANY)],
    out_specs=pl.BlockSpec(memory_space=pltpu.VMEM),
    out_shape=jax.ShapeDtypeStruct((_NT, 8, _D), jnp.float32),
    scratch_shapes=[
        pltpu.VMEM((_NBUF, 2, _RC, _D), jnp.float32),
        pltpu.SMEM((_NP,), jnp.int32),
        pltpu.SemaphoreType.DMA,
        pltpu.SemaphoreType.DMA,
        pltpu.SemaphoreType.DMA,
        pltpu.SemaphoreType.DMA,
    ],
)


def _mlp_body(p_ref, s_ref, l1_ref, l2_ref,
              w1_ref, b1_ref, w2_ref, b2_ref, o_ref):
    s = jnp.sum(s_ref[...], axis=1)
    s1_tail = s[_B:]
    e1_sum = (p_ref[0] + p_ref[1]) + jnp.concatenate(
        [jnp.zeros((_NSC, _D), jnp.float32), s1_tail], axis=0)
    e1 = e1_sum / l1_ref[...]
    e2 = s[:_B] / l2_ref[...]
    f = jnp.concatenate([e1, e2, jnp.abs(e1 - e2), e1 * e2], axis=1)
    h = jnp.dot(f, w1_ref[...], preferred_element_type=jnp.float32)
    h = jnp.maximum(h + b1_ref[...], 0.0)
    o = jnp.dot(h, w2_ref[...], preferred_element_type=jnp.float32)
    o_ref[...] = o + b2_ref[...]


_mlp = pl.pallas_call(
    _mlp_body,
    out_shape=jax.ShapeDtypeStruct((_B, _O), jnp.float32),
)


def kernel(X1, x1_lengths, X2, x2_lengths, W1, b1, W2, b2):
    sc_mask = jnp.arange(_B, dtype=jnp.int32) < _NSC
    l1_sc = jnp.where(sc_mask, x1_lengths, 0)
    p1 = _pool(X1, l1_sc)
    lens_tc = jnp.concatenate([x2_lengths, x1_lengths[_NSC:]])
    s = _tcpool(lens_tc, X2, X1)
    l1f = x1_lengths.astype(jnp.float32).reshape(_B, 1)
    l2f = x2_lengths.astype(jnp.float32).reshape(_B, 1)
    return _mlp(p1, s, l1f, l2f,
                W1, b1.reshape(1, _H), W2, b2.reshape(1, _O))
